# Initial kernel scaffold; baseline (speedup 1.0000x reference)
import jax
import jax.numpy as jnp
from jax.experimental import pallas as pl


def kernel(x, random_projection, codebook):
    raise NotImplementedError("write your pallas kernel here")


# speedup vs baseline: 1.9326x; 1.9326x over previous
import jax
import jax.numpy as jnp
from jax.experimental import pallas as pl

_TOK_BLK = 256


def _rpq_kernel(x_ref, p_ref, cbt_ref, out_ref):
    x = x_ref[...]
    p = p_ref[...]
    cbt = cbt_ref[...]
    proj = jnp.dot(x, p, preferred_element_type=jnp.float32)
    xn = proj / jnp.maximum(
        jnp.sqrt(jnp.sum(proj * proj, axis=1, keepdims=True)), 1e-12)
    cbn = cbt / jnp.maximum(
        jnp.sqrt(jnp.sum(cbt * cbt, axis=0, keepdims=True)), 1e-12)
    cb_sq = jnp.sum(cbn * cbn, axis=0, keepdims=True)
    cross = jnp.dot(xn, cbn, preferred_element_type=jnp.float32)
    s = cb_sq - 2.0 * cross
    out_ref[0, 0, :] = jnp.argmin(s, axis=1).astype(jnp.int32)


def kernel(x, random_projection, codebook):
    b, n, d = x.shape
    k, e = codebook.shape
    flat = x.reshape(b * n, d)
    cbt = codebook.T
    g = (b * n) // _TOK_BLK
    out = pl.pallas_call(
        _rpq_kernel,
        grid=(g,),
        in_specs=[
            pl.BlockSpec((_TOK_BLK, d), lambda i: (i, 0)),
            pl.BlockSpec((d, e), lambda i: (0, 0)),
            pl.BlockSpec((e, k), lambda i: (0, 0)),
        ],
        out_specs=pl.BlockSpec((1, 1, _TOK_BLK), lambda i: (i, 0, 0)),
        out_shape=jax.ShapeDtypeStruct((g, 1, _TOK_BLK), jnp.int32),
    )(flat, random_projection, cbt)
    return out.reshape(b, n)


# speedup vs baseline: 2.8628x; 1.4813x over previous
import jax
import jax.numpy as jnp
from jax.experimental import pallas as pl
from jax.experimental.pallas import tpu as pltpu

_TOK_BLK = 512
_KAUG = 40


def _rpq_kernel(x_ref, p_ref, cbt_ref, out_ref, a_ref):
    @pl.when(pl.program_id(0) == 0)
    def _build_codebook():
        cbt = cbt_ref[...]
        cbn = cbt / jnp.maximum(
            jnp.sqrt(jnp.sum(cbt * cbt, axis=0, keepdims=True)), 1e-12)
        cb_sq = jnp.sum(cbn * cbn, axis=0, keepdims=True)
        a_ref[0:32, :] = -2.0 * cbn
        a_ref[32:33, :] = cb_sq
        a_ref[33:_KAUG, :] = jnp.zeros((_KAUG - 33, cbt.shape[1]), jnp.float32)

    x = x_ref[...]
    p = p_ref[...]
    proj = jnp.dot(x, p, preferred_element_type=jnp.float32)
    xn = proj / jnp.maximum(
        jnp.sqrt(jnp.sum(proj * proj, axis=1, keepdims=True)), 1e-12)
    t = xn.shape[0]
    xn1 = jnp.concatenate(
        [xn, jnp.ones((t, 1), jnp.float32),
         jnp.zeros((t, _KAUG - 33), jnp.float32)], axis=1)
    s = jnp.dot(xn1, a_ref[...], preferred_element_type=jnp.float32)
    out_ref[0, 0, :] = jnp.argmin(s, axis=1).astype(jnp.int32)


def kernel(x, random_projection, codebook):
    b, n, d = x.shape
    k, e = codebook.shape
    flat = x.reshape(b * n, d)
    cbt = codebook.T
    g = (b * n) // _TOK_BLK
    out = pl.pallas_call(
        _rpq_kernel,
        grid=(g,),
        in_specs=[
            pl.BlockSpec((_TOK_BLK, d), lambda i: (i, 0)),
            pl.BlockSpec((d, e), lambda i: (0, 0)),
            pl.BlockSpec((e, k), lambda i: (0, 0)),
        ],
        out_specs=pl.BlockSpec((1, 1, _TOK_BLK), lambda i: (i, 0, 0)),
        out_shape=jax.ShapeDtypeStruct((g, 1, _TOK_BLK), jnp.int32),
        scratch_shapes=[pltpu.VMEM((_KAUG, k), jnp.float32)],
    )(flat, random_projection, cbt)
    return out.reshape(b, n)
